# Initial kernel scaffold; baseline (speedup 1.0000x reference)
import jax
import jax.numpy as jnp
from jax.experimental import pallas as pl


def kernel(x, Wih_first, Wih_rest, Whh, bih, bhh, Wlw, blw, Wsel, bsel):
    raise NotImplementedError("write your pallas kernel here")


# speedup vs baseline: 9.4323x; 9.4323x over previous
import jax
import jax.numpy as jnp
from jax.experimental import pallas as pl
from jax.experimental.pallas import tpu as pltpu

_IN, _H, _L, _S = 512, 512, 2, 3
_B, _T = 64, 512
_PENALTY = 0.7
_TC = 32
_G3 = 3 * _H


def _body(xb_ref, wlw_ref, wsel_ref, bsel_ref, blw_ref,
          wih0_ref, whh0_ref, bih0_ref, bhh0_ref,
          wih1_ref, whh1_ref, bih1_ref, bhh1_ref,
          out_ref, h_ref, p_ref):
    t_base = pl.program_id(0) * _TC

    @pl.when(t_base == 0)
    def _init():
        h_ref[...] = jnp.zeros((2 * _B, _H), jnp.float32)
        lane = jax.lax.broadcasted_iota(jnp.int32, (1, 128), 1)
        p_ref[...] = jnp.where(lane < _S, 1.0, 0.0).astype(jnp.float32)

    def step(tl, carry):
        t = t_base + tl
        x_t = xb_ref[pl.ds(tl, 1)].reshape(_B, _IN)
        h2d = h_ref[...]
        h_bf = h2d.astype(jnp.bfloat16)

        energy = jnp.dot(h_bf, wlw_ref[...], preferred_element_type=jnp.float32)
        le = jnp.sum(energy + blw_ref[...], axis=-1, keepdims=True)
        le_f = le.astype(jnp.bfloat16).astype(jnp.float32)
        prod = le_f * h_bf.astype(jnp.float32)
        sh = (prod[:_B] + prod[_B:]).astype(jnp.bfloat16)
        logits = (jnp.dot(sh, wsel_ref[:_H], preferred_element_type=jnp.float32)
                  + jnp.dot(x_t, wsel_ref[_H:], preferred_element_type=jnp.float32)
                  + bsel_ref[...])
        slog = jnp.sum(logits, axis=0, keepdims=True)
        m = jnp.max(slog, axis=-1, keepdims=True)
        e = jnp.exp(slog - m)
        probs = (e / jnp.sum(e, axis=-1, keepdims=True)) * p_ref[...]
        cur = jnp.where(t == 0, 0, jnp.argmax(probs)).astype(jnp.int32)

        lane = jax.lax.broadcasted_iota(jnp.int32, (1, 128), 1)
        pn = p_ref[...] * jnp.where(lane == cur, _PENALTY, 1.0)
        p_ref[...] = pn / jnp.max(pn)

        h0 = h2d[:_B]
        h1 = h2d[_B:]

        def cell(xin_bf, h_prev, wih, whh, bih, bhh):
            gi = jnp.dot(xin_bf, wih, preferred_element_type=jnp.float32) + bih
            gh = jnp.dot(h_prev.astype(jnp.bfloat16), whh,
                         preferred_element_type=jnp.float32) + bhh
            r = jax.nn.sigmoid(gi[:, :_H] + gh[:, :_H])
            z = jax.nn.sigmoid(gi[:, _H:2 * _H] + gh[:, _H:2 * _H])
            n = jnp.tanh(gi[:, 2 * _H:] + r * gh[:, 2 * _H:])
            return (1.0 - z) * n + z * h_prev

        w = lambda ref: ref[pl.ds(cur, 1)].reshape(_IN, _G3)
        b = lambda ref: ref[pl.ds(cur, 1)].reshape(1, _G3)
        h0n = cell(x_t, h0, w(wih0_ref), w(whh0_ref), b(bih0_ref), b(bhh0_ref))
        h1n = cell(h0n.astype(jnp.bfloat16), h1,
                   w(wih1_ref), w(whh1_ref), b(bih1_ref), b(bhh1_ref))

        h_ref[:_B] = h0n
        h_ref[_B:] = h1n
        out_ref[pl.ds(tl, 1)] = h1n.reshape(1, _B, _H)
        return carry

    jax.lax.fori_loop(0, _TC, step, 0)


def kernel(x, Wih_first, Wih_rest, Whh, bih, bhh, Wlw, blw, Wsel, bsel):
    f32, bf16 = jnp.float32, jnp.bfloat16
    xb = jnp.swapaxes(x, 0, 1).astype(bf16)
    wlw_t = Wlw.T.astype(bf16)
    wsel_t = jnp.zeros((_H + _IN, 128), f32).at[:, :_S].set(Wsel.T).astype(bf16)
    bsel_p = jnp.full((1, 128), -jnp.inf, f32).at[0, :_S].set(bsel)
    blw_r = blw.reshape(1, _H)
    wih0 = Wih_first.transpose(0, 2, 1).astype(bf16)
    wih1 = Wih_rest[:, 0].transpose(0, 2, 1).astype(bf16)
    whh0 = Whh[:, 0].transpose(0, 2, 1).astype(bf16)
    whh1 = Whh[:, 1].transpose(0, 2, 1).astype(bf16)
    bih0, bih1 = bih[:, 0][:, None, :], bih[:, 1][:, None, :]
    bhh0, bhh1 = bhh[:, 0][:, None, :], bhh[:, 1][:, None, :]

    full = lambda a: pl.BlockSpec(a.shape, lambda i: (0,) * a.ndim)
    outputs = pl.pallas_call(
        _body,
        grid=(_T // _TC,),
        in_specs=[pl.BlockSpec((_TC, _B, _IN), lambda i: (i, 0, 0))]
        + [full(a) for a in (wlw_t, wsel_t, bsel_p, blw_r,
                             wih0, whh0, bih0, bhh0, wih1, whh1, bih1, bhh1)],
        out_specs=pl.BlockSpec((_TC, _B, _H), lambda i: (i, 0, 0)),
        out_shape=jax.ShapeDtypeStruct((_T, _B, _H), f32),
        scratch_shapes=[pltpu.VMEM((2 * _B, _H), f32),
                        pltpu.VMEM((1, 128), f32)],
        compiler_params=pltpu.CompilerParams(
            dimension_semantics=("arbitrary",),
            vmem_limit_bytes=64 * 1024 * 1024,
        ),
    )(xb, wlw_t, wsel_t, bsel_p, blw_r,
      wih0, whh0, bih0, bhh0, wih1, whh1, bih1, bhh1)
    return outputs, outputs[-1]
